# Initial kernel scaffold; baseline (speedup 1.0000x reference)
import jax
import jax.numpy as jnp
from jax.experimental import pallas as pl


def kernel(x, edge_index, edge_attr, global_features, batch, params):
    raise NotImplementedError("write your pallas kernel here")


# speedup vs baseline: 2.6814x; 2.6814x over previous
import functools
import math

import jax
import jax.numpy as jnp
from jax import lax
from jax.experimental import pallas as pl
from jax.experimental.pallas import tpu as pltpu
from jax.experimental.pallas import tpu_sc as plsc

_LN2 = 0.6931471805599453
NC, NS = 2, 16
NW = NC * NS
CHUNK = 128
FIRE = 8
E_TILE = 3200


def _sp2(x):
    return jax.nn.softplus(x) - _LN2


def _dot(a, b):
    return jnp.dot(a, b, preferred_element_type=jnp.float32)


def _init_body(x_ref, gf_ref, wn1, bn1, wn2, bn2, wg1, bg1, wg2, bg2,
               v0_ref, u0_ref):
    v = _sp2(_dot(x_ref[...], wn1[...]) + bn1[...])
    v0_ref[...] = _sp2(_dot(v, wn2[...]) + bn2[...])
    u = _sp2(_dot(gf_ref[...], wg1[...]) + bg1[...])
    u0_ref[...] = _sp2(_dot(u, wg2[...]) + bg2[...])


def _edge_init_body(ea_ref, w1, b1, w2, b2, e0_ref):
    h = _sp2(_dot(ea_ref[...], w1[...]) + b1[...])
    e0_ref[...] = _sp2(_dot(h, w2[...]) + b2[...])


def _pre_body(v_ref, u_ref, wn1, bn1, wn2, bn2, wg1, bg1, wg2, bg2,
              w1eu, b1e, v2_ref, u2_ref, ebias_ref):
    h = _sp2(_dot(v_ref[...], wn1[...]) + bn1[...])
    v2_ref[...] = _sp2(_dot(h, wn2[...]) + bn2[...])
    u = _sp2(_dot(u_ref[...], wg1[...]) + bg1[...])
    u2 = _sp2(_dot(u, wg2[...]) + bg2[...])
    u2_ref[...] = u2
    ebias_ref[...] = _dot(u2, w1eu[...]) + b1e[...]


def _edge_mlp_body(ep_ref, vs_ref, vd_ref, ebias,
                   wpe1, bpe1, wpe2, bpe2,
                   w1s, w1d, w1e, w2, b2, w3, b3,
                   en_ref, enext_ref, esum_ref):
    i = pl.program_id(0)
    ep = ep_ref[...]
    e2 = _sp2(_dot(ep, wpe1[...]) + bpe1[...])
    e2 = _sp2(_dot(e2, wpe2[...]) + bpe2[...])
    h = _sp2(_dot(vs_ref[...], w1s[...]) + _dot(vd_ref[...], w1d[...])
             + _dot(e2, w1e[...]) + ebias[...])
    h = _sp2(_dot(h, w2[...]) + b2[...])
    en = _sp2(_dot(h, w3[...]) + b3[...])
    en_ref[...] = en
    enext_ref[...] = ep + en
    part = jnp.sum(en, axis=0, keepdims=True)

    @pl.when(i == 0)
    def _():
        esum_ref[...] = part

    @pl.when(i > 0)
    def _():
        esum_ref[...] = esum_ref[...] + part


def _node_body(vp_ref, v2_ref, acc_ref, cnt_ref, u_ref, u2_ref, esum_ref,
               w1v, w1a, w1u, b1, w2, b2, w3, b3,
               g1v, g1e, g1u, gb1, g2, gb2, g3, gb3,
               vn_ref, un_ref, *, n_nodes, n_edges):
    a = acc_ref[0, :n_nodes, :] + acc_ref[1, :n_nodes, :]
    c = cnt_ref[0, :n_nodes, 0:1] + cnt_ref[1, :n_nodes, 0:1]
    agg = a / jnp.clip(c, 1.0, None)
    v2 = v2_ref[...]
    u2 = u2_ref[...]
    h = _sp2(_dot(v2, w1v[...]) + _dot(agg, w1a[...]) + _dot(u2, w1u[...])
             + b1[...])
    h = _sp2(_dot(h, w2[...]) + b2[...])
    nv = _sp2(_dot(h, w3[...]) + b3[...])
    vn_ref[...] = vp_ref[...] + nv
    mean_v = jnp.sum(nv, axis=0, keepdims=True) * (1.0 / n_nodes)
    mean_e = esum_ref[...] * (1.0 / n_edges)
    g = _sp2(_dot(mean_v, g1v[...]) + _dot(mean_e, g1e[...])
             + _dot(u2, g1u[...]) + gb1[...])
    g = _sp2(_dot(g, g2[...]) + gb2[...])
    g = _sp2(_dot(g, g3[...]) + gb3[...])
    un_ref[...] = u_ref[...] + g


def _head_masks(n2, nh):
    dh = n2 // nh
    d_i = lax.broadcasted_iota(jnp.int32, (n2, nh), 0)
    h_i = lax.broadcasted_iota(jnp.int32, (n2, nh), 1)
    return (d_i // dh == h_i).astype(jnp.float32)


def _pool_nodes_body(vf_ref, uf_ref, wq, bq, wk, bk, wv, bv, wo, bo, out_ref):
    n2 = wq.shape[0]
    nh = 4
    dh = n2 // nh
    msk = _head_masks(n2, nh)
    q = _dot(uf_ref[...], wq[...]) + bq[...]
    k = _dot(vf_ref[...], wk[...]) + bk[...]
    vv = _dot(vf_ref[...], wv[...]) + bv[...]
    s = _dot(k * q, msk) * (1.0 / math.sqrt(dh))
    m = jnp.max(s, axis=0, keepdims=True)
    p = jnp.exp(s - m)
    l = jnp.sum(p, axis=0, keepdims=True)
    pn = p / l
    acc = lax.dot_general(vv, pn, (((0,), (0,)), ((), ())),
                          preferred_element_type=jnp.float32)
    out32 = jnp.sum(acc * msk, axis=1, keepdims=True)
    out_ref[...] = lax.dot_general(
        out32, wo[...], (((0,), (0,)), ((), ())),
        preferred_element_type=jnp.float32) + bo[...]


def _pool_edges_body(ef_ref, uf_ref, wq, bq, wk, bk, wv, bv, wo, bo,
                     out_ref, m_s, l_s, acc_s, *, ntiles):
    i = pl.program_id(0)
    n2 = wq.shape[0]
    nh = 4
    dh = n2 // nh
    msk = _head_masks(n2, nh)

    @pl.when(i == 0)
    def _():
        m_s[...] = jnp.full((1, nh), -1e30, jnp.float32)
        l_s[...] = jnp.zeros((1, nh), jnp.float32)
        acc_s[...] = jnp.zeros((n2, nh), jnp.float32)

    ef = ef_ref[...]
    q = _dot(uf_ref[...], wq[...]) + bq[...]
    k = _dot(ef, wk[...]) + bk[...]
    vv = _dot(ef, wv[...]) + bv[...]
    s = _dot(k * q, msk) * (1.0 / math.sqrt(dh))
    mt = jnp.max(s, axis=0, keepdims=True)
    m_old = m_s[...]
    m_new = jnp.maximum(m_old, mt)
    alpha = jnp.exp(m_old - m_new)
    p = jnp.exp(s - m_new)
    m_s[...] = m_new
    l_s[...] = l_s[...] * alpha + jnp.sum(p, axis=0, keepdims=True)
    acc_s[...] = acc_s[...] * alpha + lax.dot_general(
        vv, p, (((0,), (0,)), ((), ())), preferred_element_type=jnp.float32)

    @pl.when(i == ntiles - 1)
    def _():
        a = acc_s[...] * msk / l_s[...]
        colsum = jnp.sum(a, axis=1, keepdims=True)
        out_ref[...] = lax.dot_general(
            colsum, wo[...], (((0,), (0,)), ((), ())),
            preferred_element_type=jnp.float32) + bo[...]


def _final_body(no_ref, eo_ref, uf_ref, w1a, w1b, w1c, b1, w2, b2, w3, b3,
                out_ref):
    h = _sp2(_dot(no_ref[...], w1a[...]) + _dot(eo_ref[...], w1b[...])
             + _dot(uf_ref[...], w1c[...]) + b1[...])
    h = _sp2(_dot(h, w2[...]) + b2[...])
    out_ref[...] = _dot(h, w3[...]) + b3[...]


def _sc_gather(table, sidx, didx, ch):
    nt, d = table.shape
    e_pad = NW * ch * CHUNK
    mesh = plsc.VectorSubcoreMesh(core_axis_name="c", subcore_axis_name="s")
    grp = FIRE * CHUNK

    def body(table_hbm, sidx_hbm, didx_hbm, vs_hbm, vd_hbm,
             sidx_v, didx_v, srows, drows, sem_s, sem_d, sem_o):
        c = lax.axis_index("c")
        s = lax.axis_index("s")
        wid = s * NC + c
        pltpu.sync_copy(sidx_hbm.at[wid], sidx_v)
        pltpu.sync_copy(didx_hbm.at[wid], didx_v)
        base_w = wid * ch * CHUNK

        def outer(g, carry):
            descs = []
            for j in range(FIRE):
                cidx = g * FIRE + j
                descs.append(pltpu.async_copy(
                    table_hbm.at[sidx_v.at[cidx]],
                    srows.at[pl.ds(j * CHUNK, CHUNK)], sem_s))
                descs.append(pltpu.async_copy(
                    table_hbm.at[didx_v.at[cidx]],
                    drows.at[pl.ds(j * CHUNK, CHUNK)], sem_d))
            for desc in descs:
                desc.wait()
            base = base_w + g * grp
            d1 = pltpu.async_copy(srows, vs_hbm.at[pl.ds(base, grp)], sem_o)
            d2 = pltpu.async_copy(drows, vd_hbm.at[pl.ds(base, grp)], sem_o)
            d1.wait()
            d2.wait()
            return carry

        lax.fori_loop(0, ch // FIRE, outer, 0)

    fn = pl.kernel(
        body,
        out_type=(jax.ShapeDtypeStruct((e_pad, d), jnp.float32),
                  jax.ShapeDtypeStruct((e_pad, d), jnp.float32)),
        mesh=mesh,
        compiler_params=pltpu.CompilerParams(use_tc_tiling_on_sc=False),
        scratch_types=(
            pltpu.VMEM((ch, CHUNK), jnp.int32),
            pltpu.VMEM((ch, CHUNK), jnp.int32),
            pltpu.VMEM((grp, d), jnp.float32),
            pltpu.VMEM((grp, d), jnp.float32),
            pltpu.SemaphoreType.DMA,
            pltpu.SemaphoreType.DMA,
            pltpu.SemaphoreType.DMA,
        ),
    )
    return fn(table, sidx, didx)


def _sc_scatter(en_pad, didx, zeros32, zeros16, ones16, ch, with_counts):
    nacc, d = zeros32.shape
    rows_per_sub = nacc // NS
    mesh = plsc.VectorSubcoreMesh(core_axis_name="c", subcore_axis_name="s")
    grp = FIRE * CHUNK

    def body(en_hbm, didx_hbm, z32_hbm, z16_hbm, ones_hbm,
             acc_out, cnt_out, idx_v, rows, ones_v, sem_in, sem_sc,
             shared_acc, shared_cnt):
        c = lax.axis_index("c")
        s = lax.axis_index("s")
        wid = s * NC + c
        sl = pl.ds(s * rows_per_sub, rows_per_sub)
        pltpu.sync_copy(z32_hbm.at[sl], shared_acc.at[sl])
        if with_counts:
            pltpu.sync_copy(z16_hbm.at[sl], shared_cnt.at[sl])
            pltpu.sync_copy(ones_hbm, ones_v)
        pltpu.sync_copy(didx_hbm.at[wid], idx_v)
        plsc.subcore_barrier()

        def outer(g, carry):
            base = wid * ch * CHUNK + g * grp
            pltpu.async_copy(en_hbm.at[pl.ds(base, grp)], rows, sem_in).wait()
            descs = []
            for j in range(FIRE):
                cidx = g * FIRE + j
                descs.append(pltpu.async_copy(
                    rows.at[pl.ds(j * CHUNK, CHUNK)],
                    shared_acc.at[idx_v.at[cidx]], sem_sc, add=True))
                if with_counts:
                    descs.append(pltpu.async_copy(
                        ones_v, shared_cnt.at[idx_v.at[cidx]], sem_sc,
                        add=True))
            for desc in descs:
                desc.wait()
            return carry

        lax.fori_loop(0, ch // FIRE, outer, 0)
        plsc.subcore_barrier()
        pltpu.sync_copy(shared_acc.at[sl], acc_out.at[c, sl])
        if with_counts:
            pltpu.sync_copy(shared_cnt.at[sl], cnt_out.at[c, sl])

    out_type = [jax.ShapeDtypeStruct((NC, nacc, d), jnp.float32),
                jax.ShapeDtypeStruct((NC, nacc, 16), jnp.float32)]

    fn = pl.kernel(
        body,
        out_type=tuple(out_type),
        mesh=mesh,
        compiler_params=pltpu.CompilerParams(use_tc_tiling_on_sc=False),
        scratch_types=(
            pltpu.VMEM((ch, CHUNK), jnp.int32),
            pltpu.VMEM((grp, d), jnp.float32),
            pltpu.VMEM((CHUNK, 16), jnp.float32),
            pltpu.SemaphoreType.DMA,
            pltpu.SemaphoreType.DMA,
            pltpu.VMEM_SHARED((nacc, d), jnp.float32),
            pltpu.VMEM_SHARED((nacc, 16), jnp.float32),
        ),
    )
    return fn(en_pad, didx, zeros32, zeros16, ones16)


def _lin2(p):
    return (p["w"], p["b"][None, :])


def _tc_call(body, out_shapes, grid=None, in_specs=None, out_specs=None,
             scratch_shapes=()):
    kwargs = {}
    if grid is not None:
        kwargs["grid"] = grid
        kwargs["in_specs"] = in_specs
        kwargs["out_specs"] = out_specs
    if scratch_shapes:
        kwargs["scratch_shapes"] = scratch_shapes
    return pl.pallas_call(body, out_shape=out_shapes, **kwargs)


def kernel(x, edge_index, edge_attr, global_features, batch, params):
    n_nodes = x.shape[0]
    n_edges = edge_index.shape[1]
    n2 = params["ff_node"][1]["w"].shape[1]

    ch = -(-n_edges // (NW * CHUNK))
    ch = -(-ch // FIRE) * FIRE
    e_pad = NW * ch * CHUNK
    nacc = n_nodes + 16
    assert n_edges % E_TILE == 0
    ntiles = n_edges // E_TILE

    src = edge_index[0]
    dst = edge_index[1]
    sidx = jnp.pad(src, (0, e_pad - n_edges)).reshape(NW, ch, CHUNK)
    didx = jnp.pad(dst, (0, e_pad - n_edges)).reshape(NW, ch, CHUNK)
    didx_s = jnp.pad(dst, (0, e_pad - n_edges),
                     constant_values=n_nodes).reshape(NW, ch, CHUNK)
    zeros32 = jnp.zeros((nacc, n2), jnp.float32)
    zeros16 = jnp.zeros((nacc, 16), jnp.float32)
    ones16 = jnp.ones((CHUNK, 16), jnp.float32)

    fn = _lin2(params["ff_node"][0]) + _lin2(params["ff_node"][1])
    fg = _lin2(params["ff_global"][0]) + _lin2(params["ff_global"][1])
    v, u = _tc_call(
        _init_body,
        (jax.ShapeDtypeStruct((n_nodes, n2), jnp.float32),
         jax.ShapeDtypeStruct((1, n2), jnp.float32)),
    )(x, global_features, *fn, *fg)

    fe = _lin2(params["ff_edge"][0]) + _lin2(params["ff_edge"][1])
    d_edge = edge_attr.shape[1]
    e = _tc_call(
        _edge_init_body,
        jax.ShapeDtypeStruct((n_edges, n2), jnp.float32),
        grid=(ntiles,),
        in_specs=[pl.BlockSpec((E_TILE, d_edge), lambda i: (i, 0))]
        + [pl.BlockSpec(w.shape, lambda i: (0, 0)) for w in fe],
        out_specs=pl.BlockSpec((E_TILE, n2), lambda i: (i, 0)),
    )(edge_attr, *fe)

    cnt = None
    for bi, bp in enumerate(params["blocks"]):
        pn = _lin2(bp["pre_node"][0]) + _lin2(bp["pre_node"][1])
        pg = _lin2(bp["pre_global"][0]) + _lin2(bp["pre_global"][1])
        pe = _lin2(bp["pre_edge"][0]) + _lin2(bp["pre_edge"][1])
        w1 = bp["edge_mlp"][0]["w"]
        b1e = bp["edge_mlp"][0]["b"][None, :]
        w1s, w1d, w1e, w1u = (w1[0:n2], w1[n2:2 * n2], w1[2 * n2:3 * n2],
                              w1[3 * n2:4 * n2])
        em2 = _lin2(bp["edge_mlp"][1])
        em3 = _lin2(bp["edge_mlp"][2])

        v2, u2, ebias = _tc_call(
            _pre_body,
            (jax.ShapeDtypeStruct((n_nodes, n2), jnp.float32),
             jax.ShapeDtypeStruct((1, n2), jnp.float32),
             jax.ShapeDtypeStruct((1, w1u.shape[1]), jnp.float32)),
        )(v, u, *pn, *pg, w1u, b1e)

        vs, vd = _sc_gather(v2, sidx, didx, ch)

        wmats = (pe[0], pe[1], pe[2], pe[3], w1s, w1d, w1e,
                 em2[0], em2[1], em3[0], em3[1])
        en_pad, e_next, esum = _tc_call(
            _edge_mlp_body,
            (jax.ShapeDtypeStruct((e_pad, n2), jnp.float32),
             jax.ShapeDtypeStruct((n_edges, n2), jnp.float32),
             jax.ShapeDtypeStruct((1, n2), jnp.float32)),
            grid=(ntiles,),
            in_specs=[pl.BlockSpec((E_TILE, n2), lambda i: (i, 0)),
                      pl.BlockSpec((E_TILE, n2), lambda i: (i, 0)),
                      pl.BlockSpec((E_TILE, n2), lambda i: (i, 0)),
                      pl.BlockSpec(ebias.shape, lambda i: (0, 0))]
            + [pl.BlockSpec(w.shape, lambda i: (0, 0)) for w in wmats],
            out_specs=(pl.BlockSpec((E_TILE, n2), lambda i: (i, 0)),
                       pl.BlockSpec((E_TILE, n2), lambda i: (i, 0)),
                       pl.BlockSpec((1, n2), lambda i: (0, 0))),
        )(e, vs, vd, ebias, *wmats)

        if bi == 0:
            acc, cnt = _sc_scatter(en_pad, didx_s, zeros32, zeros16, ones16,
                                   ch, True)
        else:
            acc, _ = _sc_scatter(en_pad, didx_s, zeros32, zeros16, ones16,
                                 ch, False)

        nw1 = bp["node_mlp"][0]["w"]
        nb1 = bp["node_mlp"][0]["b"][None, :]
        n1v, n1a, n1u = nw1[0:n2], nw1[n2:2 * n2], nw1[2 * n2:3 * n2]
        nm2 = _lin2(bp["node_mlp"][1])
        nm3 = _lin2(bp["node_mlp"][2])
        gw1 = bp["global_mlp"][0]["w"]
        gb1 = bp["global_mlp"][0]["b"][None, :]
        g1v, g1e, g1u = gw1[0:n2], gw1[n2:2 * n2], gw1[2 * n2:3 * n2]
        gm2 = _lin2(bp["global_mlp"][1])
        gm3 = _lin2(bp["global_mlp"][2])

        v, u = _tc_call(
            functools.partial(_node_body, n_nodes=n_nodes, n_edges=n_edges),
            (jax.ShapeDtypeStruct((n_nodes, n2), jnp.float32),
             jax.ShapeDtypeStruct((1, n2), jnp.float32)),
        )(v, v2, acc, cnt, u, u2, esum,
          n1v, n1a, n1u, nb1, *nm2, *nm3,
          g1v, g1e, g1u, gb1, *gm2, *gm3)
        e = e_next

    ap = params["pool_nodes"]
    pw = (_lin2(ap["wq"]) + _lin2(ap["wk"]) + _lin2(ap["wv"])
          + _lin2(ap["wo"]))
    node_out = _tc_call(
        _pool_nodes_body,
        jax.ShapeDtypeStruct((1, n2), jnp.float32),
    )(v, u, *pw)

    ap = params["pool_edges"]
    pw = (_lin2(ap["wq"]) + _lin2(ap["wk"]) + _lin2(ap["wv"])
          + _lin2(ap["wo"]))
    edge_out = _tc_call(
        functools.partial(_pool_edges_body, ntiles=ntiles),
        jax.ShapeDtypeStruct((1, n2), jnp.float32),
        grid=(ntiles,),
        in_specs=[pl.BlockSpec((E_TILE, n2), lambda i: (i, 0)),
                  pl.BlockSpec((1, n2), lambda i: (0, 0))]
        + [pl.BlockSpec(w.shape, lambda i: (0, 0)) for w in pw],
        out_specs=pl.BlockSpec((1, n2), lambda i: (0, 0)),
        scratch_shapes=(pltpu.VMEM((1, 4), jnp.float32),
                        pltpu.VMEM((1, 4), jnp.float32),
                        pltpu.VMEM((n2, 4), jnp.float32)),
    )(e, u, *pw)

    dw1 = params["dense"][0]["w"]
    db1 = params["dense"][0]["b"][None, :]
    d1a, d1b, d1c = dw1[0:n2], dw1[n2:2 * n2], dw1[2 * n2:3 * n2]
    d2 = _lin2(params["dense"][1])
    d3 = _lin2(params["out"])
    out = _tc_call(
        _final_body,
        jax.ShapeDtypeStruct((1, 1), jnp.float32),
    )(node_out, edge_out, u, d1a, d1b, d1c, db1, *d2, *d3)
    return out


# speedup vs baseline: 5.4159x; 2.0198x over previous
import functools
import math

import jax
import jax.numpy as jnp
from jax import lax
from jax.experimental import pallas as pl
from jax.experimental.pallas import tpu as pltpu
from jax.experimental.pallas import tpu_sc as plsc

_LN2 = 0.6931471805599453
NC, NS = 2, 16
NW = NC * NS
CHUNK = 128
FIRE = 8
PK = 4
E_TILE_P = 800


def _sp2(x):
    return jax.nn.softplus(x) - _LN2


def _dot(a, b):
    return jnp.dot(a, b, preferred_element_type=jnp.float32)


def _bd(w):
    return jnp.kron(jnp.eye(PK, dtype=w.dtype), w)


def _tile4(b):
    return jnp.tile(b[None, :], (1, PK))


def _init_body(x_ref, gf_ref, wn1, bn1, wn2, bn2, wg1, bg1, wg2, bg2,
               v0_ref, u0_ref):
    v = _sp2(_dot(x_ref[...], wn1[...]) + bn1[...])
    v0_ref[...] = _sp2(_dot(v, wn2[...]) + bn2[...])
    u = _sp2(_dot(gf_ref[...], wg1[...]) + bg1[...])
    u0_ref[...] = _sp2(_dot(u, wg2[...]) + bg2[...])


def _edge_init_body(ea_ref, w1, b1, w2, b2, e0_ref):
    h = _sp2(_dot(ea_ref[...], w1[...]) + b1[...])
    e0_ref[...] = _sp2(_dot(h, w2[...]) + b2[...])


def _pre_body(v_ref, u_ref, wn1, bn1, wn2, bn2, wg1, bg1, wg2, bg2,
              w1eu, b1e, v2_ref, u2_ref, ebias_ref):
    h = _sp2(_dot(v_ref[...], wn1[...]) + bn1[...])
    v2_ref[...] = _sp2(_dot(h, wn2[...]) + bn2[...])
    u = _sp2(_dot(u_ref[...], wg1[...]) + bg1[...])
    u2 = _sp2(_dot(u, wg2[...]) + bg2[...])
    u2_ref[...] = u2
    ebias_ref[...] = _dot(u2, w1eu[...]) + b1e[...]


def _edge_mlp_body(ep_ref, vs_ref, vd_ref, ebias,
                   wpe1, bpe1, wpe2, bpe2,
                   w1sde, w2, b2, w3, b3,
                   en_ref, enext_ref, esum_ref):
    i = pl.program_id(0)
    ep = ep_ref[...]
    e2 = _sp2(_dot(ep, wpe1[...]) + bpe1[...])
    e2 = _sp2(_dot(e2, wpe2[...]) + bpe2[...])
    cat = jnp.concatenate([vs_ref[...], vd_ref[...], e2], axis=1)
    h = _sp2(_dot(cat, w1sde[...]) + ebias[...])
    h = _sp2(_dot(h, w2[...]) + b2[...])
    en = _sp2(_dot(h, w3[...]) + b3[...])
    en_ref[...] = en
    enext_ref[...] = ep + en
    part = jnp.sum(en, axis=0, keepdims=True)

    @pl.when(i == 0)
    def _():
        esum_ref[...] = part

    @pl.when(i > 0)
    def _():
        esum_ref[...] = esum_ref[...] + part


def _node_body(vp_ref, v2_ref, acc_ref, cnt_ref, u_ref, u2_ref, esum_ref,
               w1v, w1a, w1u, b1, w2, b2, w3, b3,
               g1v, g1e, g1u, gb1, g2, gb2, g3, gb3,
               vn_ref, un_ref, *, n_nodes, n_edges, n2):
    a = acc_ref[0, :n_nodes, :] + acc_ref[1, :n_nodes, :]
    c = cnt_ref[0, :n_nodes, 0:1] + cnt_ref[1, :n_nodes, 0:1]
    agg = a / jnp.clip(c, 1.0, None)
    v2 = v2_ref[...]
    u2 = u2_ref[...]
    h = _sp2(_dot(v2, w1v[...]) + _dot(agg, w1a[...]) + _dot(u2, w1u[...])
             + b1[...])
    h = _sp2(_dot(h, w2[...]) + b2[...])
    nv = _sp2(_dot(h, w3[...]) + b3[...])
    vn_ref[...] = vp_ref[...] + nv
    mean_v = jnp.sum(nv, axis=0, keepdims=True) * (1.0 / n_nodes)
    ep = esum_ref[...]
    esum32 = (ep[:, 0:n2] + ep[:, n2:2 * n2] + ep[:, 2 * n2:3 * n2]
              + ep[:, 3 * n2:4 * n2])
    mean_e = esum32 * (1.0 / n_edges)
    g = _sp2(_dot(mean_v, g1v[...]) + _dot(mean_e, g1e[...])
             + _dot(u2, g1u[...]) + gb1[...])
    g = _sp2(_dot(g, g2[...]) + gb2[...])
    g = _sp2(_dot(g, g3[...]) + gb3[...])
    un_ref[...] = u_ref[...] + g


def _head_mask(n2, nh):
    dh = n2 // nh
    d_i = lax.broadcasted_iota(jnp.int32, (n2, nh), 0)
    h_i = lax.broadcasted_iota(jnp.int32, (n2, nh), 1)
    return (d_i // dh == h_i).astype(jnp.float32)


def _slot_head_mask(n2, nh):
    dh = n2 // nh
    d_i = lax.broadcasted_iota(jnp.int32, (PK * n2, PK * nh), 0)
    c_i = lax.broadcasted_iota(jnp.int32, (PK * n2, PK * nh), 1)
    return ((d_i // n2 == c_i // nh)
            & ((d_i % n2) // dh == c_i % nh)).astype(jnp.float32)


def _pool_nodes_body(vf_ref, uf_ref, wq, bq, wk, bk, wv, bv, wo, bo, out_ref):
    n2 = wq.shape[0]
    nh = 4
    dh = n2 // nh
    msk = _head_mask(n2, nh)
    q = _dot(uf_ref[...], wq[...]) + bq[...]
    k = _dot(vf_ref[...], wk[...]) + bk[...]
    vv = _dot(vf_ref[...], wv[...]) + bv[...]
    s = _dot(k * q, msk) * (1.0 / math.sqrt(dh))
    m = jnp.max(s, axis=0, keepdims=True)
    p = jnp.exp(s - m)
    l = jnp.sum(p, axis=0, keepdims=True)
    pn = p / l
    acc = lax.dot_general(vv, pn, (((0,), (0,)), ((), ())),
                          preferred_element_type=jnp.float32)
    out32 = jnp.sum(acc * msk, axis=1, keepdims=True)
    out_ref[...] = lax.dot_general(
        out32, wo[...], (((0,), (0,)), ((), ())),
        preferred_element_type=jnp.float32) + bo[...]


def _pool_edges_body(ef_ref, uf_ref, wq, bq, wkp, bkp, wvp, bvp, wo, bo,
                     out_ref, m_s, l_s, acc_s, *, ntiles, n2):
    i = pl.program_id(0)
    nh = 4
    dh = n2 // nh
    msk = _slot_head_mask(n2, nh)

    @pl.when(i == 0)
    def _():
        m_s[...] = jnp.full((1, PK * nh), -1e30, jnp.float32)
        l_s[...] = jnp.zeros((1, PK * nh), jnp.float32)
        acc_s[...] = jnp.zeros((PK * n2, PK * nh), jnp.float32)

    ef = ef_ref[...]
    q = _dot(uf_ref[...], wq[...]) + bq[...]
    qt = jnp.concatenate([q] * PK, axis=1)
    k = _dot(ef, wkp[...]) + bkp[...]
    vv = _dot(ef, wvp[...]) + bvp[...]
    s = _dot(k * qt, msk) * (1.0 / math.sqrt(dh))
    mt = jnp.max(s, axis=0, keepdims=True)
    m_old = m_s[...]
    m_new = jnp.maximum(m_old, mt)
    alpha = jnp.exp(m_old - m_new)
    p = jnp.exp(s - m_new)
    m_s[...] = m_new
    l_s[...] = l_s[...] * alpha + jnp.sum(p, axis=0, keepdims=True)
    acc_s[...] = acc_s[...] * alpha + lax.dot_general(
        vv, p, (((0,), (0,)), ((), ())), preferred_element_type=jnp.float32)

    @pl.when(i == ntiles - 1)
    def _():
        m_f = m_s[...]
        l_f = l_s[...]
        acc_f = acc_s[...]
        mh = m_f[:, 0:nh]
        for sl in range(1, PK):
            mh = jnp.maximum(mh, m_f[:, sl * nh:(sl + 1) * nh])
        lh = jnp.zeros((1, nh), jnp.float32)
        o32 = jnp.zeros((n2, nh), jnp.float32)
        for sl in range(PK):
            corr = jnp.exp(m_f[:, sl * nh:(sl + 1) * nh] - mh)
            lh = lh + l_f[:, sl * nh:(sl + 1) * nh] * corr
            o32 = o32 + acc_f[sl * n2:(sl + 1) * n2,
                              sl * nh:(sl + 1) * nh] * corr
        o32 = o32 / lh
        hm = _head_mask(n2, nh)
        colsum = jnp.sum(o32 * hm, axis=1, keepdims=True)
        out_ref[...] = lax.dot_general(
            colsum, wo[...], (((0,), (0,)), ((), ())),
            preferred_element_type=jnp.float32) + bo[...]


def _final_body(no_ref, eo_ref, uf_ref, w1a, w1b, w1c, b1, w2, b2, w3, b3,
                out_ref):
    h = _sp2(_dot(no_ref[...], w1a[...]) + _dot(eo_ref[...], w1b[...])
             + _dot(uf_ref[...], w1c[...]) + b1[...])
    h = _sp2(_dot(h, w2[...]) + b2[...])
    out_ref[...] = _dot(h, w3[...]) + b3[...]


def _sc_gather(table, sidx, didx, ch):
    nt, d = table.shape
    e_pad = NW * ch * CHUNK
    mesh = plsc.VectorSubcoreMesh(core_axis_name="c", subcore_axis_name="s")
    grp = FIRE * CHUNK

    def body(table_hbm, sidx_hbm, didx_hbm, vs_hbm, vd_hbm,
             sidx_v, didx_v, srows, drows, sem_s, sem_d, sem_o):
        c = lax.axis_index("c")
        s = lax.axis_index("s")
        wid = s * NC + c
        pltpu.sync_copy(sidx_hbm.at[wid], sidx_v)
        pltpu.sync_copy(didx_hbm.at[wid], didx_v)
        base_w = wid * ch * CHUNK

        def outer(g, carry):
            descs = []
            for j in range(FIRE):
                cidx = g * FIRE + j
                descs.append(pltpu.async_copy(
                    table_hbm.at[sidx_v.at[cidx]],
                    srows.at[pl.ds(j * CHUNK, CHUNK)], sem_s))
                descs.append(pltpu.async_copy(
                    table_hbm.at[didx_v.at[cidx]],
                    drows.at[pl.ds(j * CHUNK, CHUNK)], sem_d))
            for desc in descs:
                desc.wait()
            base = base_w + g * grp
            d1 = pltpu.async_copy(srows, vs_hbm.at[pl.ds(base, grp)], sem_o)
            d2 = pltpu.async_copy(drows, vd_hbm.at[pl.ds(base, grp)], sem_o)
            d1.wait()
            d2.wait()
            return carry

        lax.fori_loop(0, ch // FIRE, outer, 0)

    fn = pl.kernel(
        body,
        out_type=(jax.ShapeDtypeStruct((e_pad, d), jnp.float32),
                  jax.ShapeDtypeStruct((e_pad, d), jnp.float32)),
        mesh=mesh,
        compiler_params=pltpu.CompilerParams(use_tc_tiling_on_sc=False),
        scratch_types=(
            pltpu.VMEM((ch, CHUNK), jnp.int32),
            pltpu.VMEM((ch, CHUNK), jnp.int32),
            pltpu.VMEM((grp, d), jnp.float32),
            pltpu.VMEM((grp, d), jnp.float32),
            pltpu.SemaphoreType.DMA,
            pltpu.SemaphoreType.DMA,
            pltpu.SemaphoreType.DMA,
        ),
    )
    return fn(table, sidx, didx)


def _sc_scatter(en_pad, didx, zeros32, zeros16, ones16, ch, with_counts):
    nacc, d = zeros32.shape
    rows_per_sub = nacc // NS
    mesh = plsc.VectorSubcoreMesh(core_axis_name="c", subcore_axis_name="s")
    grp = FIRE * CHUNK

    def body(en_hbm, didx_hbm, z32_hbm, z16_hbm, ones_hbm,
             acc_out, cnt_out, idx_v, rows, ones_v, sem_in, sem_sc,
             shared_acc, shared_cnt):
        c = lax.axis_index("c")
        s = lax.axis_index("s")
        wid = s * NC + c
        sl = pl.ds(s * rows_per_sub, rows_per_sub)
        pltpu.sync_copy(z32_hbm.at[sl], shared_acc.at[sl])
        if with_counts:
            pltpu.sync_copy(z16_hbm.at[sl], shared_cnt.at[sl])
            pltpu.sync_copy(ones_hbm, ones_v)
        pltpu.sync_copy(didx_hbm.at[wid], idx_v)
        plsc.subcore_barrier()

        def outer(g, carry):
            base = wid * ch * CHUNK + g * grp
            pltpu.async_copy(en_hbm.at[pl.ds(base, grp)], rows, sem_in).wait()
            descs = []
            for j in range(FIRE):
                cidx = g * FIRE + j
                descs.append(pltpu.async_copy(
                    rows.at[pl.ds(j * CHUNK, CHUNK)],
                    shared_acc.at[idx_v.at[cidx]], sem_sc, add=True))
                if with_counts:
                    descs.append(pltpu.async_copy(
                        ones_v, shared_cnt.at[idx_v.at[cidx]], sem_sc,
                        add=True))
            for desc in descs:
                desc.wait()
            return carry

        lax.fori_loop(0, ch // FIRE, outer, 0)
        plsc.subcore_barrier()
        pltpu.sync_copy(shared_acc.at[sl], acc_out.at[c, sl])
        if with_counts:
            pltpu.sync_copy(shared_cnt.at[sl], cnt_out.at[c, sl])

    out_type = [jax.ShapeDtypeStruct((NC, nacc, d), jnp.float32),
                jax.ShapeDtypeStruct((NC, nacc, 16), jnp.float32)]

    fn = pl.kernel(
        body,
        out_type=tuple(out_type),
        mesh=mesh,
        compiler_params=pltpu.CompilerParams(use_tc_tiling_on_sc=False),
        scratch_types=(
            pltpu.VMEM((ch, CHUNK), jnp.int32),
            pltpu.VMEM((grp, d), jnp.float32),
            pltpu.VMEM((CHUNK, 16), jnp.float32),
            pltpu.SemaphoreType.DMA,
            pltpu.SemaphoreType.DMA,
            pltpu.VMEM_SHARED((nacc, d), jnp.float32),
            pltpu.VMEM_SHARED((nacc, 16), jnp.float32),
        ),
    )
    return fn(en_pad, didx, zeros32, zeros16, ones16)


def _lin2(p):
    return (p["w"], p["b"][None, :])


def _lin_packed(p):
    return (_bd(p["w"]), _tile4(p["b"]))


def _tc_call(body, out_shapes, grid=None, in_specs=None, out_specs=None,
             scratch_shapes=()):
    kwargs = {}
    if grid is not None:
        kwargs["grid"] = grid
        kwargs["in_specs"] = in_specs
        kwargs["out_specs"] = out_specs
    if scratch_shapes:
        kwargs["scratch_shapes"] = scratch_shapes
    return pl.pallas_call(body, out_shape=out_shapes, **kwargs)


def kernel(x, edge_index, edge_attr, global_features, batch, params):
    n_nodes = x.shape[0]
    n_edges = edge_index.shape[1]
    n2 = params["ff_node"][1]["w"].shape[1]
    n2p = PK * n2

    ch = -(-n_edges // (NW * CHUNK))
    ch = -(-ch // FIRE) * FIRE
    e_pad = NW * ch * CHUNK
    nacc = n_nodes + 16
    assert n_edges % (PK * E_TILE_P) == 0
    ntiles = n_edges // (PK * E_TILE_P)
    ep_rows = n_edges // PK

    src = edge_index[0]
    dst = edge_index[1]
    sidx = jnp.pad(src, (0, e_pad - n_edges)).reshape(NW, ch, CHUNK)
    didx = jnp.pad(dst, (0, e_pad - n_edges)).reshape(NW, ch, CHUNK)
    didx_s = jnp.pad(dst, (0, e_pad - n_edges),
                     constant_values=n_nodes).reshape(NW, ch, CHUNK)
    zeros32 = jnp.zeros((nacc, n2), jnp.float32)
    zeros16 = jnp.zeros((nacc, 16), jnp.float32)
    ones16 = jnp.ones((CHUNK, 16), jnp.float32)

    fn = _lin2(params["ff_node"][0]) + _lin2(params["ff_node"][1])
    fg = _lin2(params["ff_global"][0]) + _lin2(params["ff_global"][1])
    v, u = _tc_call(
        _init_body,
        (jax.ShapeDtypeStruct((n_nodes, n2), jnp.float32),
         jax.ShapeDtypeStruct((1, n2), jnp.float32)),
    )(x, global_features, *fn, *fg)

    fe = _lin_packed(params["ff_edge"][0]) + _lin_packed(params["ff_edge"][1])
    d_edge = edge_attr.shape[1]
    ea_p = edge_attr.reshape(ep_rows, PK * d_edge)
    e = _tc_call(
        _edge_init_body,
        jax.ShapeDtypeStruct((ep_rows, n2p), jnp.float32),
        grid=(ntiles,),
        in_specs=[pl.BlockSpec((E_TILE_P, PK * d_edge), lambda i: (i, 0))]
        + [pl.BlockSpec(w.shape, lambda i: (0, 0)) for w in fe],
        out_specs=pl.BlockSpec((E_TILE_P, n2p), lambda i: (i, 0)),
    )(ea_p, *fe)

    cnt = None
    for bi, bp in enumerate(params["blocks"]):
        pn = _lin2(bp["pre_node"][0]) + _lin2(bp["pre_node"][1])
        pg = _lin2(bp["pre_global"][0]) + _lin2(bp["pre_global"][1])
        pe = _lin_packed(bp["pre_edge"][0]) + _lin_packed(bp["pre_edge"][1])
        w1 = bp["edge_mlp"][0]["w"]
        b1e = bp["edge_mlp"][0]["b"][None, :]
        w1sde = jnp.concatenate(
            [_bd(w1[0:n2]), _bd(w1[n2:2 * n2]), _bd(w1[2 * n2:3 * n2])],
            axis=0)
        w1u = w1[3 * n2:4 * n2]
        em2 = _lin_packed(bp["edge_mlp"][1])
        em3 = _lin_packed(bp["edge_mlp"][2])

        v2, u2, ebias = _tc_call(
            _pre_body,
            (jax.ShapeDtypeStruct((n_nodes, n2), jnp.float32),
             jax.ShapeDtypeStruct((1, n2), jnp.float32),
             jax.ShapeDtypeStruct((1, w1u.shape[1]), jnp.float32)),
        )(v, u, *pn, *pg, w1u, b1e)
        ebias_p = jnp.tile(ebias, (1, PK))

        vs, vd = _sc_gather(v2, sidx, didx, ch)
        vs_p = vs.reshape(e_pad // PK, n2p)
        vd_p = vd.reshape(e_pad // PK, n2p)

        wmats = (pe[0], pe[1], pe[2], pe[3], w1sde,
                 em2[0], em2[1], em3[0], em3[1])
        n1p = PK * bp["edge_mlp"][1]["w"].shape[0]
        en_p, e_next, esum_p = _tc_call(
            _edge_mlp_body,
            (jax.ShapeDtypeStruct((e_pad // PK, n2p), jnp.float32),
             jax.ShapeDtypeStruct((ep_rows, n2p), jnp.float32),
             jax.ShapeDtypeStruct((1, n2p), jnp.float32)),
            grid=(ntiles,),
            in_specs=[pl.BlockSpec((E_TILE_P, n2p), lambda i: (i, 0)),
                      pl.BlockSpec((E_TILE_P, n2p), lambda i: (i, 0)),
                      pl.BlockSpec((E_TILE_P, n2p), lambda i: (i, 0)),
                      pl.BlockSpec((1, n1p), lambda i: (0, 0))]
            + [pl.BlockSpec(w.shape, lambda i: (0, 0)) for w in wmats],
            out_specs=(pl.BlockSpec((E_TILE_P, n2p), lambda i: (i, 0)),
                       pl.BlockSpec((E_TILE_P, n2p), lambda i: (i, 0)),
                       pl.BlockSpec((1, n2p), lambda i: (0, 0))),
        )(e, vs_p, vd_p, ebias_p, *wmats)

        en32 = en_p.reshape(e_pad, n2)
        if bi == 0:
            acc, cnt = _sc_scatter(en32, didx_s, zeros32, zeros16, ones16,
                                   ch, True)
        else:
            acc, _ = _sc_scatter(en32, didx_s, zeros32, zeros16, ones16,
                                 ch, False)

        nw1 = bp["node_mlp"][0]["w"]
        nb1 = bp["node_mlp"][0]["b"][None, :]
        n1v, n1a, n1u = nw1[0:n2], nw1[n2:2 * n2], nw1[2 * n2:3 * n2]
        nm2 = _lin2(bp["node_mlp"][1])
        nm3 = _lin2(bp["node_mlp"][2])
        gw1 = bp["global_mlp"][0]["w"]
        gb1 = bp["global_mlp"][0]["b"][None, :]
        g1v, g1e, g1u = gw1[0:n2], gw1[n2:2 * n2], gw1[2 * n2:3 * n2]
        gm2 = _lin2(bp["global_mlp"][1])
        gm3 = _lin2(bp["global_mlp"][2])

        v, u = _tc_call(
            functools.partial(_node_body, n_nodes=n_nodes, n_edges=n_edges,
                              n2=n2),
            (jax.ShapeDtypeStruct((n_nodes, n2), jnp.float32),
             jax.ShapeDtypeStruct((1, n2), jnp.float32)),
        )(v, v2, acc, cnt, u, u2, esum_p,
          n1v, n1a, n1u, nb1, *nm2, *nm3,
          g1v, g1e, g1u, gb1, *gm2, *gm3)
        e = e_next

    ap = params["pool_nodes"]
    pw = (_lin2(ap["wq"]) + _lin2(ap["wk"]) + _lin2(ap["wv"])
          + _lin2(ap["wo"]))
    node_out = _tc_call(
        _pool_nodes_body,
        jax.ShapeDtypeStruct((1, n2), jnp.float32),
    )(v, u, *pw)

    ap = params["pool_edges"]
    pw = (_lin2(ap["wq"]) + _lin_packed(ap["wk"]) + _lin_packed(ap["wv"])
          + _lin2(ap["wo"]))
    edge_out = _tc_call(
        functools.partial(_pool_edges_body, ntiles=ntiles, n2=n2),
        jax.ShapeDtypeStruct((1, n2), jnp.float32),
        grid=(ntiles,),
        in_specs=[pl.BlockSpec((E_TILE_P, n2p), lambda i: (i, 0)),
                  pl.BlockSpec((1, n2), lambda i: (0, 0))]
        + [pl.BlockSpec(w.shape, lambda i: (0, 0)) for w in pw],
        out_specs=pl.BlockSpec((1, n2), lambda i: (0, 0)),
        scratch_shapes=(pltpu.VMEM((1, PK * 4), jnp.float32),
                        pltpu.VMEM((1, PK * 4), jnp.float32),
                        pltpu.VMEM((n2p, PK * 4), jnp.float32)),
    )(e, u, *pw)

    dw1 = params["dense"][0]["w"]
    db1 = params["dense"][0]["b"][None, :]
    d1a, d1b, d1c = dw1[0:n2], dw1[n2:2 * n2], dw1[2 * n2:3 * n2]
    d2 = _lin2(params["dense"][1])
    d3 = _lin2(params["out"])
    out = _tc_call(
        _final_body,
        jax.ShapeDtypeStruct((1, 1), jnp.float32),
    )(node_out, edge_out, u, d1a, d1b, d1c, db1, *d2, *d3)
    return out


# speedup vs baseline: 5.6228x; 1.0382x over previous
import functools
import math

import jax
import jax.numpy as jnp
from jax import lax
from jax.experimental import pallas as pl
from jax.experimental.pallas import tpu as pltpu
from jax.experimental.pallas import tpu_sc as plsc

_LN2 = 0.6931471805599453
NC, NS = 2, 16
NW = NC * NS
CHUNK = 128
FIRE = 4
NSLICE = 4
PK = 4
E_TILE_P = 800


def _sp2(x):
    return jax.nn.softplus(x) - _LN2


def _dot(a, b):
    return jnp.dot(a, b, preferred_element_type=jnp.float32)


def _bd(w):
    return jnp.kron(jnp.eye(PK, dtype=w.dtype), w)


def _tile4(b):
    return jnp.tile(b[None, :], (1, PK))


def _init_body(x_ref, gf_ref, wn1, bn1, wn2, bn2, wg1, bg1, wg2, bg2,
               v0_ref, u0_ref):
    v = _sp2(_dot(x_ref[...], wn1[...]) + bn1[...])
    v0_ref[...] = _sp2(_dot(v, wn2[...]) + bn2[...])
    u = _sp2(_dot(gf_ref[...], wg1[...]) + bg1[...])
    u0_ref[...] = _sp2(_dot(u, wg2[...]) + bg2[...])


def _edge_init_body(ea_ref, w1, b1, w2, b2, e0_ref):
    h = _sp2(_dot(ea_ref[...], w1[...]) + b1[...])
    e0_ref[...] = _sp2(_dot(h, w2[...]) + b2[...])


def _pre_body(v_ref, u_ref, wn1, bn1, wn2, bn2, wg1, bg1, wg2, bg2,
              w1eu, b1e, v2_ref, u2_ref, ebias_ref):
    h = _sp2(_dot(v_ref[...], wn1[...]) + bn1[...])
    v2_ref[...] = _sp2(_dot(h, wn2[...]) + bn2[...])
    u = _sp2(_dot(u_ref[...], wg1[...]) + bg1[...])
    u2 = _sp2(_dot(u, wg2[...]) + bg2[...])
    u2_ref[...] = u2
    ebias_ref[...] = _dot(u2, w1eu[...]) + b1e[...]


def _edge_mlp_body(ep_ref, vs_ref, vd_ref, ebias,
                   wpe1, bpe1, wpe2, bpe2,
                   w1sde, w2, b2, w3, b3,
                   en_ref, enext_ref, esum_ref):
    i = pl.program_id(0)
    ep = ep_ref[...]
    e2 = _sp2(_dot(ep, wpe1[...]) + bpe1[...])
    e2 = _sp2(_dot(e2, wpe2[...]) + bpe2[...])
    cat = jnp.concatenate([vs_ref[...], vd_ref[...], e2], axis=1)
    h = _sp2(_dot(cat, w1sde[...]) + ebias[...])
    h = _sp2(_dot(h, w2[...]) + b2[...])
    en = _sp2(_dot(h, w3[...]) + b3[...])
    en_ref[...] = en
    enext_ref[...] = ep + en
    part = jnp.sum(en, axis=0, keepdims=True)

    @pl.when(i == 0)
    def _():
        esum_ref[...] = part

    @pl.when(i > 0)
    def _():
        esum_ref[...] = esum_ref[...] + part


def _node_body(vp_ref, v2_ref, acc_ref, cnt_ref, u_ref, u2_ref,
               es0, es1, es2, es3,
               w1v, w1a, w1u, b1, w2, b2, w3, b3,
               g1v, g1e, g1u, gb1, g2, gb2, g3, gb3,
               vn_ref, un_ref, *, n_nodes, n_edges, n2):
    a = acc_ref[0, :n_nodes, :] + acc_ref[1, :n_nodes, :]
    c = cnt_ref[0, :n_nodes, 0:1] + cnt_ref[1, :n_nodes, 0:1]
    agg = a / jnp.clip(c, 1.0, None)
    v2 = v2_ref[...]
    u2 = u2_ref[...]
    h = _sp2(_dot(v2, w1v[...]) + _dot(agg, w1a[...]) + _dot(u2, w1u[...])
             + b1[...])
    h = _sp2(_dot(h, w2[...]) + b2[...])
    nv = _sp2(_dot(h, w3[...]) + b3[...])
    vn_ref[...] = vp_ref[...] + nv
    mean_v = jnp.sum(nv, axis=0, keepdims=True) * (1.0 / n_nodes)
    ep = es0[...] + es1[...] + es2[...] + es3[...]
    esum32 = (ep[:, 0:n2] + ep[:, n2:2 * n2] + ep[:, 2 * n2:3 * n2]
              + ep[:, 3 * n2:4 * n2])
    mean_e = esum32 * (1.0 / n_edges)
    g = _sp2(_dot(mean_v, g1v[...]) + _dot(mean_e, g1e[...])
             + _dot(u2, g1u[...]) + gb1[...])
    g = _sp2(_dot(g, g2[...]) + gb2[...])
    g = _sp2(_dot(g, g3[...]) + gb3[...])
    un_ref[...] = u_ref[...] + g


def _head_mask(n2, nh):
    dh = n2 // nh
    d_i = lax.broadcasted_iota(jnp.int32, (n2, nh), 0)
    h_i = lax.broadcasted_iota(jnp.int32, (n2, nh), 1)
    return (d_i // dh == h_i).astype(jnp.float32)


def _slot_head_mask(n2, nh):
    dh = n2 // nh
    d_i = lax.broadcasted_iota(jnp.int32, (PK * n2, PK * nh), 0)
    c_i = lax.broadcasted_iota(jnp.int32, (PK * n2, PK * nh), 1)
    return ((d_i // n2 == c_i // nh)
            & ((d_i % n2) // dh == c_i % nh)).astype(jnp.float32)


def _pool_nodes_body(vf_ref, uf_ref, wq, bq, wk, bk, wv, bv, wo, bo, out_ref):
    n2 = wq.shape[0]
    nh = 4
    dh = n2 // nh
    msk = _head_mask(n2, nh)
    q = _dot(uf_ref[...], wq[...]) + bq[...]
    k = _dot(vf_ref[...], wk[...]) + bk[...]
    vv = _dot(vf_ref[...], wv[...]) + bv[...]
    s = _dot(k * q, msk) * (1.0 / math.sqrt(dh))
    m = jnp.max(s, axis=0, keepdims=True)
    p = jnp.exp(s - m)
    l = jnp.sum(p, axis=0, keepdims=True)
    pn = p / l
    acc = lax.dot_general(vv, pn, (((0,), (0,)), ((), ())),
                          preferred_element_type=jnp.float32)
    out32 = jnp.sum(acc * msk, axis=1, keepdims=True)
    out_ref[...] = lax.dot_general(
        out32, wo[...], (((0,), (0,)), ((), ())),
        preferred_element_type=jnp.float32) + bo[...]


def _pool_edges_body(ef_ref, uf_ref, wq, bq, wkp, bkp, wvp, bvp, wo, bo,
                     mi_ref, li_ref, ai_ref,
                     mo_ref, lo_ref, ao_ref, out_ref, m_s, l_s, acc_s,
                     *, ntiles, n2):
    i = pl.program_id(0)
    nh = 4
    dh = n2 // nh
    msk = _slot_head_mask(n2, nh)

    @pl.when(i == 0)
    def _():
        m_s[...] = mi_ref[...]
        l_s[...] = li_ref[...]
        acc_s[...] = ai_ref[...]

    ef = ef_ref[...]
    q = _dot(uf_ref[...], wq[...]) + bq[...]
    qt = jnp.concatenate([q] * PK, axis=1)
    k = _dot(ef, wkp[...]) + bkp[...]
    vv = _dot(ef, wvp[...]) + bvp[...]
    s = _dot(k * qt, msk) * (1.0 / math.sqrt(dh))
    mt = jnp.max(s, axis=0, keepdims=True)
    m_old = m_s[...]
    m_new = jnp.maximum(m_old, mt)
    alpha = jnp.exp(m_old - m_new)
    p = jnp.exp(s - m_new)
    m_s[...] = m_new
    l_s[...] = l_s[...] * alpha + jnp.sum(p, axis=0, keepdims=True)
    acc_s[...] = acc_s[...] * alpha + lax.dot_general(
        vv, p, (((0,), (0,)), ((), ())), preferred_element_type=jnp.float32)

    @pl.when(i == ntiles - 1)
    def _():
        m_f = m_s[...]
        l_f = l_s[...]
        acc_f = acc_s[...]
        mo_ref[...] = m_f
        lo_ref[...] = l_f
        ao_ref[...] = acc_f
        mh = m_f[:, 0:nh]
        for sl in range(1, PK):
            mh = jnp.maximum(mh, m_f[:, sl * nh:(sl + 1) * nh])
        lh = jnp.zeros((1, nh), jnp.float32)
        o32 = jnp.zeros((n2, nh), jnp.float32)
        for sl in range(PK):
            corr = jnp.exp(m_f[:, sl * nh:(sl + 1) * nh] - mh)
            lh = lh + l_f[:, sl * nh:(sl + 1) * nh] * corr
            o32 = o32 + acc_f[sl * n2:(sl + 1) * n2,
                              sl * nh:(sl + 1) * nh] * corr
        o32 = o32 / lh
        hm = _head_mask(n2, nh)
        colsum = jnp.sum(o32 * hm, axis=1, keepdims=True)
        out_ref[...] = lax.dot_general(
            colsum, wo[...], (((0,), (0,)), ((), ())),
            preferred_element_type=jnp.float32) + bo[...]


def _final_body(no_ref, eo_ref, uf_ref, w1a, w1b, w1c, b1, w2, b2, w3, b3,
                out_ref):
    h = _sp2(_dot(no_ref[...], w1a[...]) + _dot(eo_ref[...], w1b[...])
             + _dot(uf_ref[...], w1c[...]) + b1[...])
    h = _sp2(_dot(h, w2[...]) + b2[...])
    out_ref[...] = _dot(h, w3[...]) + b3[...]


def _sc_gather(table, sidx, didx, ch):
    nt, d = table.shape
    e_pad = NW * ch * CHUNK
    mesh = plsc.VectorSubcoreMesh(core_axis_name="c", subcore_axis_name="s")
    grp = FIRE * CHUNK

    def body(table_hbm, sidx_hbm, didx_hbm, vs_hbm, vd_hbm,
             sidx_v, didx_v, srows, drows, sem_s, sem_d, sem_o):
        c = lax.axis_index("c")
        s = lax.axis_index("s")
        wid = s * NC + c
        pltpu.sync_copy(sidx_hbm.at[wid], sidx_v)
        pltpu.sync_copy(didx_hbm.at[wid], didx_v)
        base_w = wid * ch * CHUNK

        def outer(g, carry):
            descs = []
            for j in range(FIRE):
                cidx = g * FIRE + j
                descs.append(pltpu.async_copy(
                    table_hbm.at[sidx_v.at[cidx]],
                    srows.at[pl.ds(j * CHUNK, CHUNK)], sem_s))
                descs.append(pltpu.async_copy(
                    table_hbm.at[didx_v.at[cidx]],
                    drows.at[pl.ds(j * CHUNK, CHUNK)], sem_d))
            for desc in descs:
                desc.wait()
            base = base_w + g * grp
            d1 = pltpu.async_copy(srows, vs_hbm.at[pl.ds(base, grp)], sem_o)
            d2 = pltpu.async_copy(drows, vd_hbm.at[pl.ds(base, grp)], sem_o)
            d1.wait()
            d2.wait()
            return carry

        lax.fori_loop(0, ch // FIRE, outer, 0)

    fn = pl.kernel(
        body,
        out_type=(jax.ShapeDtypeStruct((e_pad, d), jnp.float32),
                  jax.ShapeDtypeStruct((e_pad, d), jnp.float32)),
        mesh=mesh,
        compiler_params=pltpu.CompilerParams(use_tc_tiling_on_sc=False),
        scratch_types=(
            pltpu.VMEM((ch, CHUNK), jnp.int32),
            pltpu.VMEM((ch, CHUNK), jnp.int32),
            pltpu.VMEM((grp, d), jnp.float32),
            pltpu.VMEM((grp, d), jnp.float32),
            pltpu.SemaphoreType.DMA,
            pltpu.SemaphoreType.DMA,
            pltpu.SemaphoreType.DMA,
        ),
    )
    return fn(table, sidx, didx)


def _sc_scatter(en_list, didx4, zeros32, zeros16, ones16, ch, with_counts):
    nacc, d = zeros32.shape
    rows_per_sub = nacc // NS
    mesh = plsc.VectorSubcoreMesh(core_axis_name="c", subcore_axis_name="s")
    grp = FIRE * CHUNK

    def body(en0, en1, en2, en3, didx_hbm, z32_hbm, z16_hbm, ones_hbm,
             acc_out, cnt_out, idx_v, rows, ones_v, sem_in, sem_sc,
             shared_acc, shared_cnt):
        c = lax.axis_index("c")
        s = lax.axis_index("s")
        wid = s * NC + c
        sl = pl.ds(s * rows_per_sub, rows_per_sub)
        pltpu.sync_copy(z32_hbm.at[sl], shared_acc.at[sl])
        if with_counts:
            pltpu.sync_copy(z16_hbm.at[sl], shared_cnt.at[sl])
            pltpu.sync_copy(ones_hbm, ones_v)
        plsc.subcore_barrier()

        for si, en_hbm in enumerate((en0, en1, en2, en3)):
            pltpu.sync_copy(didx_hbm.at[si, wid], idx_v)

            def outer(g, carry, en_hbm=en_hbm):
                base = wid * ch * CHUNK + g * grp
                pltpu.async_copy(en_hbm.at[pl.ds(base, grp)], rows,
                                 sem_in).wait()
                descs = []
                for j in range(FIRE):
                    cidx = g * FIRE + j
                    descs.append(pltpu.async_copy(
                        rows.at[pl.ds(j * CHUNK, CHUNK)],
                        shared_acc.at[idx_v.at[cidx]], sem_sc, add=True))
                    if with_counts:
                        descs.append(pltpu.async_copy(
                            ones_v, shared_cnt.at[idx_v.at[cidx]], sem_sc,
                            add=True))
                for desc in descs:
                    desc.wait()
                return carry

            lax.fori_loop(0, ch // FIRE, outer, 0)

        plsc.subcore_barrier()
        pltpu.sync_copy(shared_acc.at[sl], acc_out.at[c, sl])
        if with_counts:
            pltpu.sync_copy(shared_cnt.at[sl], cnt_out.at[c, sl])

    out_type = [jax.ShapeDtypeStruct((NC, nacc, d), jnp.float32),
                jax.ShapeDtypeStruct((NC, nacc, 16), jnp.float32)]

    fn = pl.kernel(
        body,
        out_type=tuple(out_type),
        mesh=mesh,
        compiler_params=pltpu.CompilerParams(use_tc_tiling_on_sc=False),
        scratch_types=(
            pltpu.VMEM((ch, CHUNK), jnp.int32),
            pltpu.VMEM((grp, d), jnp.float32),
            pltpu.VMEM((CHUNK, 16), jnp.float32),
            pltpu.SemaphoreType.DMA,
            pltpu.SemaphoreType.DMA,
            pltpu.VMEM_SHARED((nacc, d), jnp.float32),
            pltpu.VMEM_SHARED((nacc, 16), jnp.float32),
        ),
    )
    return fn(*en_list, didx4, zeros32, zeros16, ones16)


def _lin2(p):
    return (p["w"], p["b"][None, :])


def _lin_packed(p):
    return (_bd(p["w"]), _tile4(p["b"]))


def _tc_call(body, out_shapes, grid=None, in_specs=None, out_specs=None,
             scratch_shapes=()):
    kwargs = {}
    if grid is not None:
        kwargs["grid"] = grid
        kwargs["in_specs"] = in_specs
        kwargs["out_specs"] = out_specs
    if scratch_shapes:
        kwargs["scratch_shapes"] = scratch_shapes
    return pl.pallas_call(body, out_shape=out_shapes, **kwargs)


def kernel(x, edge_index, edge_attr, global_features, batch, params):
    n_nodes = x.shape[0]
    n_edges = edge_index.shape[1]
    n2 = params["ff_node"][1]["w"].shape[1]
    n2p = PK * n2

    assert n_edges % NSLICE == 0
    es = n_edges // NSLICE
    ch = -(-es // (NW * CHUNK))
    ch = -(-ch // FIRE) * FIRE
    e_pad_s = NW * ch * CHUNK
    nacc = n_nodes + 16
    assert es % (PK * E_TILE_P) == 0
    ntiles_s = es // (PK * E_TILE_P)
    ep_rows_s = es // PK

    src = edge_index[0]
    dst = edge_index[1]

    def _slice_idx(idx, fill):
        r = idx.reshape(NSLICE, es)
        r = jnp.pad(r, ((0, 0), (0, e_pad_s - es)), constant_values=fill)
        return r.reshape(NSLICE, NW, ch, CHUNK)

    sidx4 = _slice_idx(src, 0)
    didx4 = _slice_idx(dst, 0)
    didx_s4 = _slice_idx(dst, n_nodes)
    zeros32 = jnp.zeros((nacc, n2), jnp.float32)
    zeros16 = jnp.zeros((nacc, 16), jnp.float32)
    ones16 = jnp.ones((CHUNK, 16), jnp.float32)

    fn = _lin2(params["ff_node"][0]) + _lin2(params["ff_node"][1])
    fg = _lin2(params["ff_global"][0]) + _lin2(params["ff_global"][1])
    v, u = _tc_call(
        _init_body,
        (jax.ShapeDtypeStruct((n_nodes, n2), jnp.float32),
         jax.ShapeDtypeStruct((1, n2), jnp.float32)),
    )(x, global_features, *fn, *fg)

    fe = _lin_packed(params["ff_edge"][0]) + _lin_packed(params["ff_edge"][1])
    d_edge = edge_attr.shape[1]
    ea_p = edge_attr.reshape(NSLICE * ep_rows_s, PK * d_edge)
    e_slices = []
    for s in range(NSLICE):
        e_slices.append(_tc_call(
            _edge_init_body,
            jax.ShapeDtypeStruct((ep_rows_s, n2p), jnp.float32),
            grid=(ntiles_s,),
            in_specs=[pl.BlockSpec((E_TILE_P, PK * d_edge),
                                   lambda i, s=s: (i + s * ntiles_s, 0))]
            + [pl.BlockSpec(w.shape, lambda i: (0, 0)) for w in fe],
            out_specs=pl.BlockSpec((E_TILE_P, n2p), lambda i: (i, 0)),
        )(ea_p, *fe))

    cnt = None
    for bi, bp in enumerate(params["blocks"]):
        pn = _lin2(bp["pre_node"][0]) + _lin2(bp["pre_node"][1])
        pg = _lin2(bp["pre_global"][0]) + _lin2(bp["pre_global"][1])
        pe = _lin_packed(bp["pre_edge"][0]) + _lin_packed(bp["pre_edge"][1])
        w1 = bp["edge_mlp"][0]["w"]
        b1e = bp["edge_mlp"][0]["b"][None, :]
        w1sde = jnp.concatenate(
            [_bd(w1[0:n2]), _bd(w1[n2:2 * n2]), _bd(w1[2 * n2:3 * n2])],
            axis=0)
        w1u = w1[3 * n2:4 * n2]
        em2 = _lin_packed(bp["edge_mlp"][1])
        em3 = _lin_packed(bp["edge_mlp"][2])

        v2, u2, ebias = _tc_call(
            _pre_body,
            (jax.ShapeDtypeStruct((n_nodes, n2), jnp.float32),
             jax.ShapeDtypeStruct((1, n2), jnp.float32),
             jax.ShapeDtypeStruct((1, w1u.shape[1]), jnp.float32)),
        )(v, u, *pn, *pg, w1u, b1e)
        ebias_p = jnp.tile(ebias, (1, PK))

        wmats = (pe[0], pe[1], pe[2], pe[3], w1sde,
                 em2[0], em2[1], em3[0], em3[1])
        n1p = PK * bp["edge_mlp"][1]["w"].shape[0]
        en32_list, e_next_slices, esum_list = [], [], []
        for s in range(NSLICE):
            vs, vd = _sc_gather(v2, sidx4[s], didx4[s], ch)
            vs_p = vs.reshape(e_pad_s // PK, n2p)
            vd_p = vd.reshape(e_pad_s // PK, n2p)
            en_p, e_next_s, esum_s = _tc_call(
                _edge_mlp_body,
                (jax.ShapeDtypeStruct((e_pad_s // PK, n2p), jnp.float32),
                 jax.ShapeDtypeStruct((ep_rows_s, n2p), jnp.float32),
                 jax.ShapeDtypeStruct((1, n2p), jnp.float32)),
                grid=(ntiles_s,),
                in_specs=[pl.BlockSpec((E_TILE_P, n2p), lambda i: (i, 0)),
                          pl.BlockSpec((E_TILE_P, n2p), lambda i: (i, 0)),
                          pl.BlockSpec((E_TILE_P, n2p), lambda i: (i, 0)),
                          pl.BlockSpec((1, n1p), lambda i: (0, 0))]
                + [pl.BlockSpec(w.shape, lambda i: (0, 0)) for w in wmats],
                out_specs=(pl.BlockSpec((E_TILE_P, n2p), lambda i: (i, 0)),
                           pl.BlockSpec((E_TILE_P, n2p), lambda i: (i, 0)),
                           pl.BlockSpec((1, n2p), lambda i: (0, 0))),
            )(e_slices[s], vs_p, vd_p, ebias_p, *wmats)
            en32_list.append(en_p.reshape(e_pad_s, n2))
            e_next_slices.append(e_next_s)
            esum_list.append(esum_s)

        if bi == 0:
            acc, cnt = _sc_scatter(en32_list, didx_s4, zeros32, zeros16,
                                   ones16, ch, True)
        else:
            acc, _ = _sc_scatter(en32_list, didx_s4, zeros32, zeros16,
                                 ones16, ch, False)

        nw1 = bp["node_mlp"][0]["w"]
        nb1 = bp["node_mlp"][0]["b"][None, :]
        n1v, n1a, n1u = nw1[0:n2], nw1[n2:2 * n2], nw1[2 * n2:3 * n2]
        nm2 = _lin2(bp["node_mlp"][1])
        nm3 = _lin2(bp["node_mlp"][2])
        gw1 = bp["global_mlp"][0]["w"]
        gb1 = bp["global_mlp"][0]["b"][None, :]
        g1v, g1e, g1u = gw1[0:n2], gw1[n2:2 * n2], gw1[2 * n2:3 * n2]
        gm2 = _lin2(bp["global_mlp"][1])
        gm3 = _lin2(bp["global_mlp"][2])

        v, u = _tc_call(
            functools.partial(_node_body, n_nodes=n_nodes, n_edges=n_edges,
                              n2=n2),
            (jax.ShapeDtypeStruct((n_nodes, n2), jnp.float32),
             jax.ShapeDtypeStruct((1, n2), jnp.float32)),
        )(v, v2, acc, cnt, u, u2, *esum_list,
          n1v, n1a, n1u, nb1, *nm2, *nm3,
          g1v, g1e, g1u, gb1, *gm2, *gm3)
        e_slices = e_next_slices

    ap = params["pool_nodes"]
    pw = (_lin2(ap["wq"]) + _lin2(ap["wk"]) + _lin2(ap["wv"])
          + _lin2(ap["wo"]))
    node_out = _tc_call(
        _pool_nodes_body,
        jax.ShapeDtypeStruct((1, n2), jnp.float32),
    )(v, u, *pw)

    ap = params["pool_edges"]
    pw = (_lin2(ap["wq"]) + _lin_packed(ap["wk"]) + _lin_packed(ap["wv"])
          + _lin2(ap["wo"]))
    nh4 = PK * 4
    m_c = jnp.full((1, nh4), -1e30, jnp.float32)
    l_c = jnp.zeros((1, nh4), jnp.float32)
    a_c = jnp.zeros((n2p, nh4), jnp.float32)
    edge_out = None
    for s in range(NSLICE):
        m_c, l_c, a_c, edge_out = _tc_call(
            functools.partial(_pool_edges_body, ntiles=ntiles_s, n2=n2),
            (jax.ShapeDtypeStruct((1, nh4), jnp.float32),
             jax.ShapeDtypeStruct((1, nh4), jnp.float32),
             jax.ShapeDtypeStruct((n2p, nh4), jnp.float32),
             jax.ShapeDtypeStruct((1, n2), jnp.float32)),
            grid=(ntiles_s,),
            in_specs=[pl.BlockSpec((E_TILE_P, n2p), lambda i: (i, 0)),
                      pl.BlockSpec((1, n2), lambda i: (0, 0))]
            + [pl.BlockSpec(w.shape, lambda i: (0, 0)) for w in pw]
            + [pl.BlockSpec((1, nh4), lambda i: (0, 0)),
               pl.BlockSpec((1, nh4), lambda i: (0, 0)),
               pl.BlockSpec((n2p, nh4), lambda i: (0, 0))],
            out_specs=(pl.BlockSpec((1, nh4), lambda i: (0, 0)),
                       pl.BlockSpec((1, nh4), lambda i: (0, 0)),
                       pl.BlockSpec((n2p, nh4), lambda i: (0, 0)),
                       pl.BlockSpec((1, n2), lambda i: (0, 0))),
            scratch_shapes=(pltpu.VMEM((1, nh4), jnp.float32),
                            pltpu.VMEM((1, nh4), jnp.float32),
                            pltpu.VMEM((n2p, nh4), jnp.float32)),
        )(e_slices[s], u, *pw, m_c, l_c, a_c)

    dw1 = params["dense"][0]["w"]
    db1 = params["dense"][0]["b"][None, :]
    d1a, d1b, d1c = dw1[0:n2], dw1[n2:2 * n2], dw1[2 * n2:3 * n2]
    d2 = _lin2(params["dense"][1])
    d3 = _lin2(params["out"])
    out = _tc_call(
        _final_body,
        jax.ShapeDtypeStruct((1, 1), jnp.float32),
    )(node_out, edge_out, u, d1a, d1b, d1c, db1, *d2, *d3)
    return out
